# baseline (speedup 1.0000x reference)
import jax
import jax.numpy as jnp
from jax.experimental import pallas as pl
from jax.experimental.pallas import tpu as pltpu
from jax.experimental.pallas import tpu_sc as plsc

NUM_CORES = 2
NUM_SUBCORES = 16
NUM_WORKERS = NUM_CORES * NUM_SUBCORES
GRP = 16


def _sc_gather(user_table, item_table, user_ids, item_ids):
    n, H = user_table.shape
    B = user_ids.shape[0]
    per_w = B // NUM_WORKERS
    mesh = plsc.VectorSubcoreMesh(core_axis_name="c", subcore_axis_name="s")

    @pl.kernel(
        out_type=jax.ShapeDtypeStruct((2 * B, H), jnp.float32),
        mesh=mesh,
        scratch_types=[pltpu.VMEM((per_w,), jnp.int32)]
        + [pltpu.SemaphoreType.DMA] * GRP,
    )
    def gather_kernel(ut_hbm, it_hbm, uid_hbm, iid_hbm, out_hbm, idx_v, *sems):
        wid = jax.lax.axis_index("s") * NUM_CORES + jax.lax.axis_index("c")
        base = wid * per_w
        for tab_hbm, ids_hbm, obase in ((ut_hbm, uid_hbm, 0),
                                        (it_hbm, iid_hbm, B)):
            pltpu.sync_copy(ids_hbm.at[pl.ds(base, per_w)], idx_v)

            @pl.loop(0, per_w // GRP)
            def _(g):
                vec = idx_v[pl.ds(g * GRP, GRP)]
                copies = []
                for k in range(GRP):
                    copies.append(pltpu.async_copy(
                        tab_hbm.at[vec[k]],
                        out_hbm.at[obase + base + g * GRP + k],
                        sems[k]))
                for c in copies:
                    c.wait()

    return gather_kernel(user_table, item_table, user_ids, item_ids)


def _mlp_body(uew_ref, iew_ref,
              u_W1, u_b1, u_W2, u_b2, u_W3, u_b3,
              i_W1, i_b1, i_W2, i_b2, i_W3, i_b3,
              p_W1, p_b1, p_W2, p_b2, out_ref):
    f32 = jnp.float32
    bf16 = jnp.bfloat16
    H = u_W1.shape[0]

    def dense(x, W, b, relu=True):
        y = jnp.dot(x.astype(bf16), W[...].astype(bf16),
                    preferred_element_type=f32) + b[...]
        return jnp.maximum(y, 0.0) if relu else y

    ue = dense(dense(dense(uew_ref[...], u_W1, u_b1), u_W2, u_b2), u_W3, u_b3)
    ie = dense(dense(dense(iew_ref[...], i_W1, i_b1), i_W2, i_b2), i_W3, i_b3)
    h = (jnp.dot(ue.astype(bf16), p_W1[:H, :].astype(bf16), preferred_element_type=f32)
         + jnp.dot(ie.astype(bf16), p_W1[H:, :].astype(bf16), preferred_element_type=f32)
         + p_b1[...])
    h = jnp.maximum(h, 0.0)
    out_ref[...] = dense(h, p_W2, p_b2, relu=False)


def kernel(user_ids, item_ids, user_table, item_table,
           u_W1, u_b1, u_W2, u_b2, u_W3, u_b3,
           i_W1, i_b1, i_W2, i_b2, i_W3, i_b3,
           p_W1, p_b1, p_W2, p_b2):
    B = user_ids.shape[0]
    H = user_table.shape[1]
    rows = _sc_gather(user_table, item_table, user_ids, item_ids)

    BLK = 2048
    full = lambda shape: pl.BlockSpec(shape, lambda i: tuple(0 for _ in shape))
    nblk = B // BLK
    preds = pl.pallas_call(
        _mlp_body,
        grid=(nblk,),
        in_specs=[
            pl.BlockSpec((BLK, H), lambda i: (i, 0)),
            pl.BlockSpec((BLK, H), lambda i, _n=nblk: (i + _n, 0)),
            full(u_W1.shape), full(u_b1.shape), full(u_W2.shape), full(u_b2.shape),
            full(u_W3.shape), full(u_b3.shape),
            full(i_W1.shape), full(i_b1.shape), full(i_W2.shape), full(i_b2.shape),
            full(i_W3.shape), full(i_b3.shape),
            full(p_W1.shape), full(p_b1.shape), full(p_W2.shape), full(p_b2.shape),
        ],
        out_specs=pl.BlockSpec((BLK, 1), lambda i: (i, 0)),
        out_shape=jax.ShapeDtypeStruct((B, 1), jnp.float32),
    )(rows, rows,
      u_W1, u_b1, u_W2, u_b2, u_W3, u_b3,
      i_W1, i_b1, i_W2, i_b2, i_W3, i_b3,
      p_W1, p_b1, p_W2, p_b2)
    return preds.reshape(-1)


# speedup vs baseline: 1.0259x; 1.0259x over previous
import jax
import jax.numpy as jnp
from jax.experimental import pallas as pl
from jax.experimental.pallas import tpu as pltpu
from jax.experimental.pallas import tpu_sc as plsc

NUM_CORES = 2
NUM_SUBCORES = 16
NUM_WORKERS = NUM_CORES * NUM_SUBCORES
CHUNK = 128
PACK_BLK = 10000


def _pack_body(u_ref, i_ref, out_ref):
    out_ref[:, : u_ref.shape[1]] = u_ref[...]
    out_ref[:, u_ref.shape[1]:] = i_ref[...]


def _pack_tables(user_table, item_table):
    n, H = user_table.shape
    return pl.pallas_call(
        _pack_body,
        grid=(n // PACK_BLK,),
        in_specs=[
            pl.BlockSpec((PACK_BLK, H), lambda i: (i, 0)),
            pl.BlockSpec((PACK_BLK, H), lambda i: (i, 0)),
        ],
        out_specs=pl.BlockSpec((PACK_BLK, 2 * H), lambda i: (i, 0)),
        out_shape=jax.ShapeDtypeStruct((n, 2 * H), jnp.float32),
    )(user_table, item_table)


def _sc_gather(packed, ids):
    n, W = packed.shape
    NB = ids.shape[0]
    per_w = NB // NUM_WORKERS
    mesh = plsc.VectorSubcoreMesh(core_axis_name="c", subcore_axis_name="s")

    @pl.kernel(
        out_type=jax.ShapeDtypeStruct((NB, W), jnp.float32),
        mesh=mesh,
        scratch_types=[
            pltpu.VMEM((per_w,), jnp.int32),
            pltpu.VMEM((CHUNK, W), jnp.float32),
            pltpu.VMEM((CHUNK, W), jnp.float32),
            pltpu.SemaphoreType.DMA,
            pltpu.SemaphoreType.DMA,
        ],
    )
    def gather_kernel(tab_hbm, ids_hbm, out_hbm, idx_v, rows_a, rows_b, sem_a, sem_b):
        wid = jax.lax.axis_index("s") * NUM_CORES + jax.lax.axis_index("c")
        base = wid * per_w
        pltpu.sync_copy(ids_hbm.at[pl.ds(base, per_w)], idx_v)

        pltpu.async_copy(tab_hbm.at[idx_v.at[pl.ds(0, CHUNK)]], rows_a, sem_a).wait()

        @pl.loop(0, per_w // CHUNK // 2)
        def _(h):
            g = h * 2
            wr_a = pltpu.async_copy(rows_a, out_hbm.at[pl.ds(base + g * CHUNK, CHUNK)], sem_a)
            gt_b = pltpu.async_copy(
                tab_hbm.at[idx_v.at[pl.ds((g + 1) * CHUNK, CHUNK)]], rows_b, sem_b)
            wr_a.wait()
            gt_b.wait()
            wr_b = pltpu.async_copy(rows_b, out_hbm.at[pl.ds(base + (g + 1) * CHUNK, CHUNK)], sem_b)
            is_last = g + 2 >= per_w // CHUNK
            nxt = jnp.where(is_last, 0, (g + 2) * CHUNK)
            gt_a = pltpu.async_copy(tab_hbm.at[idx_v.at[pl.ds(nxt, CHUNK)]], rows_a, sem_a)
            wr_b.wait()
            gt_a.wait()

    return gather_kernel(packed, ids)


def _mlp_body(uew_ref, iew_ref,
              u_W1, u_b1, u_W2, u_b2, u_W3, u_b3,
              i_W1, i_b1, i_W2, i_b2, i_W3, i_b3,
              p_W1, p_b1, p_W2, p_b2, out_ref):
    f32 = jnp.float32
    bf16 = jnp.bfloat16
    H = u_W1.shape[0]

    def dense(x, W, b, relu=True):
        y = jnp.dot(x.astype(bf16), W[...].astype(bf16),
                    preferred_element_type=f32) + b[...]
        return jnp.maximum(y, 0.0) if relu else y

    ue = uew_ref[:, :H]
    ie = iew_ref[:, H:]
    ue = dense(dense(dense(ue, u_W1, u_b1), u_W2, u_b2), u_W3, u_b3)
    ie = dense(dense(dense(ie, i_W1, i_b1), i_W2, i_b2), i_W3, i_b3)
    h = (jnp.dot(ue.astype(bf16), p_W1[:H, :].astype(bf16), preferred_element_type=f32)
         + jnp.dot(ie.astype(bf16), p_W1[H:, :].astype(bf16), preferred_element_type=f32)
         + p_b1[...])
    h = jnp.maximum(h, 0.0)
    out_ref[...] = dense(h, p_W2, p_b2, relu=False)


def kernel(user_ids, item_ids, user_table, item_table,
           u_W1, u_b1, u_W2, u_b2, u_W3, u_b3,
           i_W1, i_b1, i_W2, i_b2, i_W3, i_b3,
           p_W1, p_b1, p_W2, p_b2):
    B = user_ids.shape[0]
    H = user_table.shape[1]
    packed = _pack_tables(user_table, item_table)
    ids = jnp.concatenate([user_ids, item_ids])
    rows = _sc_gather(packed, ids)

    BLK = 2048
    full = lambda shape: pl.BlockSpec(shape, lambda i: tuple(0 for _ in shape))
    nblk = B // BLK
    preds = pl.pallas_call(
        _mlp_body,
        grid=(nblk,),
        in_specs=[
            pl.BlockSpec((BLK, 2 * H), lambda i: (i, 0)),
            pl.BlockSpec((BLK, 2 * H), lambda i, _n=nblk: (i + _n, 0)),
            full(u_W1.shape), full(u_b1.shape), full(u_W2.shape), full(u_b2.shape),
            full(u_W3.shape), full(u_b3.shape),
            full(i_W1.shape), full(i_b1.shape), full(i_W2.shape), full(i_b2.shape),
            full(i_W3.shape), full(i_b3.shape),
            full(p_W1.shape), full(p_b1.shape), full(p_W2.shape), full(p_b2.shape),
        ],
        out_specs=pl.BlockSpec((BLK, 1), lambda i: (i, 0)),
        out_shape=jax.ShapeDtypeStruct((B, 1), jnp.float32),
    )(rows, rows,
      u_W1, u_b1, u_W2, u_b2, u_W3, u_b3,
      i_W1, i_b1, i_W2, i_b2, i_W3, i_b3,
      p_W1, p_b1, p_W2, p_b2)
    return preds.reshape(-1)
